# Initial kernel scaffold; baseline (speedup 1.0000x reference)
import jax
import jax.numpy as jnp
from jax.experimental import pallas as pl


def kernel(x, expert_W, expert_b, gate_W, gate_b):
    raise NotImplementedError("write your pallas kernel here")


# speedup vs baseline: 2.1015x; 2.1015x over previous
import jax
import jax.numpy as jnp
from jax.experimental import pallas as pl
from jax.experimental.pallas import tpu as pltpu

E = 8
T = 2048
D = 1024
GATE_PAD = 128


def _moe_body(x_ref, W_ref, b_ref, gW_ref, gb_ref, out_ref, idx_ref):
    e = pl.program_id(0)

    @pl.when(e == 0)
    def _():
        gate = jnp.dot(
            x_ref[...], gW_ref[...],
            preferred_element_type=jnp.float32,
        ) + gb_ref[...]
        mx = jnp.max(gate, axis=1, keepdims=True)
        ii = jax.lax.broadcasted_iota(jnp.int32, gate.shape, 1)
        idx_ref[...] = jnp.min(
            jnp.where(gate == mx, ii, GATE_PAD), axis=1, keepdims=True
        )

    m = (idx_ref[...] == e).astype(jnp.float32)
    y = jax.lax.dot_general(
        x_ref[...], W_ref[0], (((1,), (1,)), ((), ())),
        preferred_element_type=jnp.float32,
    ) + b_ref[0]

    @pl.when(e == 0)
    def _():
        out_ref[...] = m * y

    @pl.when(e > 0)
    def _():
        out_ref[...] += m * y


def kernel(x, expert_W, expert_b, gate_W, gate_b):
    gWp = jnp.zeros((D, GATE_PAD), jnp.float32).at[:, :E].set(gate_W)
    gbp = jnp.full((1, GATE_PAD), -1e30, jnp.float32).at[0, :E].set(gate_b)

    return pl.pallas_call(
        _moe_body,
        grid=(E,),
        in_specs=[
            pl.BlockSpec((T, D), lambda e: (0, 0)),
            pl.BlockSpec((1, D, D), lambda e: (e, 0, 0)),
            pl.BlockSpec((1, 1, D), lambda e: (e, 0, 0)),
            pl.BlockSpec((D, GATE_PAD), lambda e: (0, 0)),
            pl.BlockSpec((1, GATE_PAD), lambda e: (0, 0)),
        ],
        out_specs=pl.BlockSpec((T, D), lambda e: (0, 0)),
        out_shape=jax.ShapeDtypeStruct((T, D), jnp.float32),
        scratch_shapes=[pltpu.VMEM((T, 1), jnp.int32)],
        compiler_params=pltpu.CompilerParams(
            dimension_semantics=("arbitrary",),
        ),
    )(x, expert_W, expert_b.reshape(E, 1, D), gWp, gbp)
